# baseline (speedup 1.0000x reference)
import jax
import jax.numpy as jnp
from jax.experimental import pallas as pl
from jax.experimental.pallas import tpu as pltpu

_N_IM2COL = 2
_PAD = (None, None, 16, 8)


def _head_body(shapes, *refs):
    n = len(shapes)
    x_refs = refs[:n]
    (wa_ref, ba_ref, ba2_ref, w1_ref, s1_ref, b1_ref, w2_ref, s2_ref,
     b2_ref, wp_ref, bp_ref) = refs[n:n + 11]
    o_refs = refs[n + 11:n + 11 + 5 * n]
    h_refs = refs[n + 11 + 5 * n:]

    C = wa_ref.shape[1]
    F = wa_ref.shape[2]
    P = wp_ref.shape[1]

    @pl.when(pl.program_id(0) == 0)
    def _():
        for h in h_refs:
            h[...] = jnp.zeros(h.shape, jnp.bfloat16)

    idn = (jax.lax.broadcasted_iota(jnp.int32, (F, F), 0) ==
           jax.lax.broadcasted_iota(jnp.int32, (F, F), 1)).astype(jnp.bfloat16)

    for i, (H, W) in enumerate(shapes):
        x_ref = x_refs[i]
        ob, oo, os_, oa, od = o_refs[5 * i:5 * i + 5]
        h0 = h_refs[2 * i]
        h1 = h_refs[2 * i + 1]
        M = H * W

        if i < _N_IM2COL:
            def to_scratch(v, dst):
                img = v.astype(jnp.bfloat16).reshape(H, W, C)
                dst[1:H + 1, 1:W + 1, 0:C] = img
                dst[1:H + 1, 0:W, C:2 * C] = img
                dst[1:H + 1, 0:W - 1, 2 * C:3 * C] = img[:, 1:, :]

            x = x_ref[0].reshape(M, C)
            a = jnp.dot(x, wa_ref[i], preferred_element_type=jnp.float32)
            a = a + ba_ref[i]

            def conv3(src_ref, w_ref, s_ref, b_ref):
                acc = None
                for di in range(3):
                    xs = src_ref[di:di + H, 0:W, :].reshape(M, 3 * C)
                    d = jnp.dot(xs, w_ref[i, di],
                                preferred_element_type=jnp.float32)
                    acc = d if acc is None else acc + d
                return jnp.maximum(acc * s_ref[i] + b_ref[i], 0.0)
        else:
            PAD = _PAD[i]
            pcol = jax.lax.broadcasted_iota(jnp.int32, (M, C), 0) % W

            def to_scratch(v, dst, pcol=pcol, PAD=PAD, M=M, W=W):
                vb = v.astype(jnp.bfloat16)
                zero = jnp.zeros_like(vb)
                variants = (jnp.where(pcol == W - 1, zero, vb), vb,
                            jnp.where(pcol == 0, zero, vb))
                for di in range(3):
                    for dj in range(3):
                        base = PAD - ((di - 1) * W + (dj - 1))
                        t = di * 3 + dj
                        dst[base:base + M, t * C:(t + 1) * C] = variants[dj]

            xb = x_ref[0].astype(jnp.bfloat16)
            aT = jax.lax.dot_general(wa_ref[i], xb,
                                     (((0,), (0,)), ((), ())),
                                     preferred_element_type=jnp.float32)
            aT = (aT + ba2_ref[i]).astype(jnp.bfloat16)
            a = jax.lax.dot_general(aT, idn, (((0,), (0,)), ((), ())),
                                    preferred_element_type=jnp.float32)

            def conv3(src_ref, w_ref, s_ref, b_ref, PAD=PAD, M=M):
                acc = None
                for di in range(3):
                    xs = src_ref[PAD:PAD + M, 3 * C * di:3 * C * (di + 1)]
                    d = jnp.dot(xs, w_ref[i, di],
                                preferred_element_type=jnp.float32)
                    acc = d if acc is None else acc + d
                return jnp.maximum(acc * s_ref[i] + b_ref[i], 0.0)

        to_scratch(a, h0)
        to_scratch(conv3(h0, w1_ref, s1_ref, b1_ref), h1)
        v2 = conv3(h1, w2_ref, s2_ref, b2_ref)

        pT = jax.lax.dot_general(wp_ref[i], v2, (((1,), (1,)), ((), ())),
                                 preferred_element_type=jnp.float32)
        pT = pT + bp_ref[i]
        k = jax.lax.broadcasted_iota(jnp.int32, (P, M), 0) % 15
        pT = jnp.where(k == 4, jax.nn.sigmoid(pT),
                       jnp.where(k == 14, jax.nn.softplus(pT) + 1.0, pT))

        for aidx in range(3):
            base = 15 * aidx
            ob[0, 4 * aidx:4 * aidx + 4] = pT[base:base + 4]
            oo[0, aidx:aidx + 1] = pT[base + 4:base + 5]
            os_[0, 5 * aidx:5 * aidx + 5] = pT[base + 5:base + 10]
            oa[0, 4 * aidx:4 * aidx + 4] = pT[base + 10:base + 14]
            od[0, aidx:aidx + 1] = pT[base + 14:base + 15]


def kernel(feat0, feat1, feat2, feat3, adapt_w, adapt_b, c1_w, c1_b, bn1_g,
           bn1_b, bn1_m, bn1_v, c2_w, c2_b, bn2_g, bn2_b, bn2_m, bn2_v,
           pred_w, pred_b):
    eps = 1e-5
    L, F = adapt_b.shape
    C = feat0.shape[1]
    P = pred_b.shape[1]
    feats = [feat0, feat1, feat2, feat3]
    bf16 = jnp.bfloat16
    B = feat0.shape[0]

    wa = adapt_w.reshape(L, F, C).transpose(0, 2, 1).astype(bf16)
    w1 = c1_w.transpose(0, 3, 4, 2, 1).reshape(L, 3, 3 * C, F).astype(bf16)
    w2 = c2_w.transpose(0, 3, 4, 2, 1).reshape(L, 3, 3 * F, F).astype(bf16)
    wp = pred_w.reshape(L, P, F)

    s1 = (bn1_g / jnp.sqrt(bn1_v + eps)).reshape(L, 1, F)
    b1 = ((c1_b - bn1_m) * s1[:, 0] + bn1_b).reshape(L, 1, F)
    s2 = (bn2_g / jnp.sqrt(bn2_v + eps)).reshape(L, 1, F)
    b2 = ((c2_b - bn2_m) * s2[:, 0] + bn2_b).reshape(L, 1, F)
    ba = adapt_b.reshape(L, 1, F)
    ba2 = adapt_b.reshape(L, F, 1)
    bp = pred_b.reshape(L, P, 1)

    shapes = []
    xs = []
    for li, f in enumerate(feats):
        _, _, H, W = f.shape
        shapes.append((H, W))
        if li < _N_IM2COL:
            xs.append(f.transpose(0, 2, 3, 1).astype(bf16))
        else:
            xs.append(f.reshape(B, C, H * W))

    full = lambda a: pl.BlockSpec(a.shape, lambda b: (0,) * a.ndim)
    in_specs = [pl.BlockSpec((1,) + x.shape[1:],
                             (lambda b: (b, 0, 0, 0)) if x.ndim == 4
                             else (lambda b: (b, 0, 0)))
                for x in xs]
    in_specs += [full(a) for a in (wa, ba, ba2, w1, s1, b1, w2, s2, b2,
                                   wp, bp)]
    out_specs = []
    out_shape = []
    for (H, W) in shapes:
        for ch in (12, 3, 15, 12, 3):
            out_specs.append(pl.BlockSpec((1, ch, H * W),
                                          lambda b: (b, 0, 0)))
            out_shape.append(
                jax.ShapeDtypeStruct((B, ch, H * W), jnp.float32))
    scratch_shapes = []
    for li, (H, W) in enumerate(shapes):
        if li < _N_IM2COL:
            W_im = -(-(W + 2) // 8) * 8
            scratch_shapes += [pltpu.VMEM((H + 2, W_im, 3 * C), bf16)] * 2
        else:
            scratch_shapes += [pltpu.VMEM((H * W + 2 * _PAD[li], 9 * C),
                                          bf16)] * 2

    ps = pl.pallas_call(
        lambda *refs: _head_body(shapes, *refs),
        grid=(B,),
        in_specs=in_specs,
        out_specs=out_specs,
        out_shape=out_shape,
        scratch_shapes=scratch_shapes,
        compiler_params=pltpu.CompilerParams(
            dimension_semantics=("arbitrary",)),
    )(*xs, wa, ba, ba2, w1, s1, b1, w2, s2, b2, wp, bp)

    outs = []
    for li, (H, W) in enumerate(shapes):
        leaves = []
        for j, ch in enumerate((12, 3, 15, 12, 3)):
            leaves.append(ps[5 * li + j].reshape(B, 3, ch // 3, H, W))
        outs.append(tuple(leaves))
    return tuple(outs)


# speedup vs baseline: 1.0319x; 1.0319x over previous
import jax
import jax.numpy as jnp
from jax.experimental import pallas as pl
from jax.experimental.pallas import tpu as pltpu


def _geom(W):
    W_o = -(-W // 8) * 8
    W_im = -(-(W_o + 2) // 8) * 8
    return W_o, W_im


def _head_body(shapes, *refs):
    n = len(shapes)
    x_refs = refs[:n]
    (wa_ref, ba_ref, w1_ref, s1_ref, b1_ref, w2_ref, s2_ref, b2_ref,
     wp_ref, bp_ref) = refs[n:n + 10]
    o_refs = refs[n + 10:n + 10 + 5 * n]
    h_refs = refs[n + 10 + 5 * n:]

    C = wa_ref.shape[1]
    F = wa_ref.shape[2]
    P = wp_ref.shape[1]

    @pl.when(pl.program_id(0) == 0)
    def _():
        for h in h_refs:
            h[...] = jnp.zeros(h.shape, jnp.bfloat16)

    for i, (H, W, W_o, W_im) in enumerate(shapes):
        x_ref = x_refs[i]
        ob, oo, os_, oa, od = o_refs[5 * i:5 * i + 5]
        h0 = h_refs[2 * i]
        h1 = h_refs[2 * i + 1]
        M = H * W_o

        if W_o != W:
            col = jax.lax.broadcasted_iota(jnp.int32, (M, F), 0) % W_o
            keep = col < W

        def to_im2col(v, dst):
            if W_o != W:
                v = jnp.where(keep, v, jnp.bfloat16(0))
            img = v.reshape(H, W_o, C)
            dst[1:H + 1, 1:W_o + 1, 0:C] = img
            dst[1:H + 1, 0:W_o, C:2 * C] = img
            dst[1:H + 1, 0:W_o - 1, 2 * C:3 * C] = img[:, 1:, :]

        x = x_ref[0].reshape(M, C)
        a = jnp.dot(x, wa_ref[i], preferred_element_type=jnp.float32)
        to_im2col((a + ba_ref[i]).astype(jnp.bfloat16), h0)

        def conv3(src_ref, w_ref, s_ref, b_ref):
            acc = None
            for di in range(3):
                xs = src_ref[di:di + H, 0:W_o, :].reshape(M, 3 * C)
                d = jnp.dot(xs, w_ref[i, di],
                            preferred_element_type=jnp.float32)
                acc = d if acc is None else acc + d
            return jnp.maximum(acc * s_ref[i] + b_ref[i],
                               0.0).astype(jnp.bfloat16)

        to_im2col(conv3(h0, w1_ref, s1_ref, b1_ref), h1)
        v2 = conv3(h1, w2_ref, s2_ref, b2_ref)

        pT = jax.lax.dot_general(wp_ref[i], v2, (((1,), (1,)), ((), ())),
                                 preferred_element_type=jnp.float32)
        pT = pT + bp_ref[i]
        k = jax.lax.broadcasted_iota(jnp.int32, (P, M), 0) % 15
        pT = jnp.where(k == 4, jax.nn.sigmoid(pT),
                       jnp.where(k == 14, jax.nn.softplus(pT) + 1.0, pT))

        for aidx in range(3):
            base = 15 * aidx
            ob[0, 4 * aidx:4 * aidx + 4] = pT[base:base + 4]
            oo[0, aidx:aidx + 1] = pT[base + 4:base + 5]
            os_[0, 5 * aidx:5 * aidx + 5] = pT[base + 5:base + 10]
            oa[0, 4 * aidx:4 * aidx + 4] = pT[base + 10:base + 14]
            od[0, aidx:aidx + 1] = pT[base + 14:base + 15]


def kernel(feat0, feat1, feat2, feat3, adapt_w, adapt_b, c1_w, c1_b, bn1_g,
           bn1_b, bn1_m, bn1_v, c2_w, c2_b, bn2_g, bn2_b, bn2_m, bn2_v,
           pred_w, pred_b):
    eps = 1e-5
    L, F = adapt_b.shape
    C = feat0.shape[1]
    P = pred_b.shape[1]
    feats = [feat0, feat1, feat2, feat3]
    bf16 = jnp.bfloat16
    B = feat0.shape[0]

    wa = adapt_w.reshape(L, F, C).transpose(0, 2, 1).astype(bf16)
    w1 = c1_w.transpose(0, 3, 4, 2, 1).reshape(L, 3, 3 * C, F).astype(bf16)
    w2 = c2_w.transpose(0, 3, 4, 2, 1).reshape(L, 3, 3 * F, F).astype(bf16)
    wp = pred_w.reshape(L, P, F)

    s1 = (bn1_g / jnp.sqrt(bn1_v + eps)).reshape(L, 1, F)
    b1 = ((c1_b - bn1_m) * s1[:, 0] + bn1_b).reshape(L, 1, F)
    s2 = (bn2_g / jnp.sqrt(bn2_v + eps)).reshape(L, 1, F)
    b2 = ((c2_b - bn2_m) * s2[:, 0] + bn2_b).reshape(L, 1, F)
    ba = adapt_b.reshape(L, 1, F)
    bp = pred_b.reshape(L, P, 1)

    shapes = []
    xs = []
    for f in feats:
        _, _, H, W = f.shape
        W_o, W_im = _geom(W)
        shapes.append((H, W, W_o, W_im))
        x = f.transpose(0, 2, 3, 1)
        if W_o != W:
            x = jnp.pad(x, ((0, 0), (0, 0), (0, W_o - W), (0, 0)))
        xs.append(x.astype(bf16))

    full = lambda a: pl.BlockSpec(a.shape, lambda b: (0,) * a.ndim)
    in_specs = (
        [pl.BlockSpec((1, H, W_o, C), lambda b: (b, 0, 0, 0))
         for (H, W, W_o, W_im) in shapes]
        + [full(a) for a in (wa, ba, w1, s1, b1, w2, s2, b2, wp, bp)]
    )
    out_specs = []
    out_shape = []
    for (H, W, W_o, W_im) in shapes:
        for ch in (12, 3, 15, 12, 3):
            out_specs.append(pl.BlockSpec((1, ch, H * W_o),
                                          lambda b: (b, 0, 0)))
            out_shape.append(
                jax.ShapeDtypeStruct((B, ch, H * W_o), jnp.float32))
    scratch_shapes = []
    for (H, W, W_o, W_im) in shapes:
        scratch_shapes += [pltpu.VMEM((H + 2, W_im, 3 * C), bf16)] * 2

    ps = pl.pallas_call(
        lambda *refs: _head_body(shapes, *refs),
        grid=(B,),
        in_specs=in_specs,
        out_specs=out_specs,
        out_shape=out_shape,
        scratch_shapes=scratch_shapes,
        compiler_params=pltpu.CompilerParams(
            dimension_semantics=("arbitrary",)),
    )(*xs, wa, ba, w1, s1, b1, w2, s2, b2, wp, bp)

    outs = []
    for li, (H, W, W_o, W_im) in enumerate(shapes):
        leaves = []
        for j, ch in enumerate((12, 3, 15, 12, 3)):
            t = ps[5 * li + j].reshape(B, 3, ch // 3, H, W_o)
            if W_o != W:
                t = t[..., :W]
            leaves.append(t)
        outs.append(tuple(leaves))
    return tuple(outs)


# speedup vs baseline: 1.0340x; 1.0020x over previous
import jax
import jax.numpy as jnp
from jax.experimental import pallas as pl
from jax.experimental.pallas import tpu as pltpu


def _geom(W):
    W_o = -(-W // 8) * 8
    W_im = -(-(W_o + 2) // 8) * 8
    return W_o, W_im


def _head_body(shapes, *refs):
    n = len(shapes)
    x_refs = refs[:n]
    (wa_ref, ba_ref, w1_ref, s1_ref, b1_ref, w2_ref, s2_ref, b2_ref,
     wp_ref, bp_ref) = refs[n:n + 10]
    o_refs = refs[n + 10:n + 10 + 5 * n]
    h_refs = refs[n + 10 + 5 * n:]

    C = wa_ref.shape[1]
    F = wa_ref.shape[2]
    P = wp_ref.shape[1]

    @pl.when(pl.program_id(0) == 0)
    def _():
        for h in h_refs:
            h[...] = jnp.zeros(h.shape, jnp.bfloat16)

    for i, (H, W, W_o, W_im) in enumerate(shapes):
        x_ref = x_refs[i]
        ob, oo, os_, oa, od = o_refs[5 * i:5 * i + 5]
        h0 = h_refs[2 * i]
        h1 = h_refs[2 * i + 1]
        M = H * W_o

        if W_o != W:
            col = jax.lax.broadcasted_iota(jnp.int32, (M, F), 0) % W_o
            keep = col < W

        def to_im2col(v, dst):
            if W_o != W:
                v = jnp.where(keep, v, jnp.bfloat16(0))
            img = v.reshape(H, W_o, C)
            dst[1:H + 1, 1:W_o + 1, 0:C] = img
            dst[1:H + 1, 0:W_o, C:2 * C] = img
            dst[1:H + 1, 0:W_o - 1, 2 * C:3 * C] = img[:, 1:, :]

        x = x_ref[0].reshape(M, C)
        a = jnp.dot(x, wa_ref[i], preferred_element_type=jnp.float32)
        to_im2col((a + ba_ref[i]).astype(jnp.bfloat16), h0)

        def conv3(src_ref, w_ref, s_ref, b_ref):
            acc = None
            for di in range(3):
                xs = src_ref[di:di + H, 0:W_o, :].reshape(M, 3 * C)
                d = jnp.dot(xs, w_ref[i, di],
                            preferred_element_type=jnp.float32)
                acc = d if acc is None else acc + d
            return jnp.maximum(acc * s_ref[i] + b_ref[i],
                               0.0).astype(jnp.bfloat16)

        to_im2col(conv3(h0, w1_ref, s1_ref, b1_ref), h1)
        v2 = conv3(h1, w2_ref, s2_ref, b2_ref)

        pT = jax.lax.dot_general(wp_ref[i], v2, (((1,), (1,)), ((), ())),
                                 preferred_element_type=jnp.float32)
        pT = pT + bp_ref[i]
        k = jax.lax.broadcasted_iota(jnp.int32, (P, M), 0) % 15
        pT = jnp.where(k == 4, jax.nn.sigmoid(pT),
                       jnp.where(k == 14, jax.nn.softplus(pT) + 1.0, pT))

        for aidx in range(3):
            base = 15 * aidx
            ob[0, 4 * aidx:4 * aidx + 4] = pT[base:base + 4]
            oo[0, aidx:aidx + 1] = pT[base + 4:base + 5]
            os_[0, 5 * aidx:5 * aidx + 5] = pT[base + 5:base + 10]
            oa[0, 4 * aidx:4 * aidx + 4] = pT[base + 10:base + 14]
            od[0, aidx:aidx + 1] = pT[base + 14:base + 15]


def kernel(feat0, feat1, feat2, feat3, adapt_w, adapt_b, c1_w, c1_b, bn1_g,
           bn1_b, bn1_m, bn1_v, c2_w, c2_b, bn2_g, bn2_b, bn2_m, bn2_v,
           pred_w, pred_b):
    eps = 1e-5
    L, F = adapt_b.shape
    C = feat0.shape[1]
    P = pred_b.shape[1]
    feats = [feat0, feat1, feat2, feat3]
    bf16 = jnp.bfloat16
    B = feat0.shape[0]

    wa = adapt_w.reshape(L, F, C).transpose(0, 2, 1).astype(bf16)
    w1 = c1_w.astype(bf16).transpose(0, 3, 4, 2, 1).reshape(L, 3, 3 * C, F)
    w2 = c2_w.astype(bf16).transpose(0, 3, 4, 2, 1).reshape(L, 3, 3 * F, F)
    wp = pred_w.reshape(L, P, F)

    s1 = (bn1_g / jnp.sqrt(bn1_v + eps)).reshape(L, 1, F)
    b1 = ((c1_b - bn1_m) * s1[:, 0] + bn1_b).reshape(L, 1, F)
    s2 = (bn2_g / jnp.sqrt(bn2_v + eps)).reshape(L, 1, F)
    b2 = ((c2_b - bn2_m) * s2[:, 0] + bn2_b).reshape(L, 1, F)
    ba = adapt_b.reshape(L, 1, F)
    bp = pred_b.reshape(L, P, 1)

    shapes = []
    xs = []
    for f in feats:
        _, _, H, W = f.shape
        W_o, W_im = _geom(W)
        shapes.append((H, W, W_o, W_im))
        x = f.transpose(0, 2, 3, 1)
        if W_o != W:
            x = jnp.pad(x, ((0, 0), (0, 0), (0, W_o - W), (0, 0)))
        xs.append(x.astype(bf16))

    full = lambda a: pl.BlockSpec(a.shape, lambda b: (0,) * a.ndim)
    in_specs = (
        [pl.BlockSpec((1, H, W_o, C), lambda b: (b, 0, 0, 0))
         for (H, W, W_o, W_im) in shapes]
        + [full(a) for a in (wa, ba, w1, s1, b1, w2, s2, b2, wp, bp)]
    )
    out_specs = []
    out_shape = []
    for (H, W, W_o, W_im) in shapes:
        for ch in (12, 3, 15, 12, 3):
            out_specs.append(pl.BlockSpec((1, ch, H * W_o),
                                          lambda b: (b, 0, 0)))
            out_shape.append(
                jax.ShapeDtypeStruct((B, ch, H * W_o), jnp.float32))
    scratch_shapes = []
    for (H, W, W_o, W_im) in shapes:
        scratch_shapes += [pltpu.VMEM((H + 2, W_im, 3 * C), bf16)] * 2

    ps = pl.pallas_call(
        lambda *refs: _head_body(shapes, *refs),
        grid=(B,),
        in_specs=in_specs,
        out_specs=out_specs,
        out_shape=out_shape,
        scratch_shapes=scratch_shapes,
        compiler_params=pltpu.CompilerParams(
            dimension_semantics=("arbitrary",)),
    )(*xs, wa, ba, w1, s1, b1, w2, s2, b2, wp, bp)

    outs = []
    for li, (H, W, W_o, W_im) in enumerate(shapes):
        leaves = []
        for j, ch in enumerate((12, 3, 15, 12, 3)):
            t = ps[5 * li + j].reshape(B, 3, ch // 3, H, W_o)
            if W_o != W:
                t = t[..., :W]
            leaves.append(t)
        outs.append(tuple(leaves))
    return tuple(outs)
